# baseline (speedup 1.0000x reference)
import functools

import jax
import jax.numpy as jnp
from jax import lax
from jax.experimental import pallas as pl
from jax.experimental.pallas import tpu as pltpu
from jax.experimental.pallas import tpu_sc as plsc

_N = 10000
_E = 320000
_D = 128
_NW = 32
_NPW = 320
_NPAD = _NW * _NPW
_EPW = _E // _NW
_C = 2000
_NCHUNK = _EPW // _C
_K = 128
_BSTRIDE = _K + 16
_NBLK = _EPW // _K + 1
_RCAP = _NBLK * _K
_CSTRIDE = 16
_DIVM = 13108
_DIVS = 22


def _tile_id():
    return lax.axis_index("s") * 2 + lax.axis_index("c")


def _sc_bin_body(src_hbm, dst_hbm, sidx_hbm, didx_hbm, cnts_hbm,
                 schunk, dchunk, bs, bd, cntb, nflb, cbuf, sem):
    wid = _tile_id()

    zero16 = jnp.zeros((16,), dtype=jnp.int32)
    trash16 = jnp.full((16,), _NPW, dtype=jnp.int32)
    for o in range(_NW):
        cntb[pl.ds(o * _CSTRIDE, 16)] = zero16
        nflb[pl.ds(o * _CSTRIDE, 16)] = zero16
        for t in range(_BSTRIDE // 16):
            bs[pl.ds(o * _BSTRIDE + t * 16, 16)] = zero16
            bd[pl.ds(o * _BSTRIDE + t * 16, 16)] = trash16

    def flush(o, nf, valid):
        base = (wid * _NW) * _RCAP + o * _RCAP + nf * _K
        pltpu.sync_copy(bs.at[pl.ds(o * _BSTRIDE, _K)],
                        sidx_hbm.at[pl.ds(base, _K)])
        pltpu.sync_copy(bd.at[pl.ds(o * _BSTRIDE, _K)],
                        didx_hbm.at[pl.ds(base, _K)])
        del valid

    def chunk_body(ch, carry):
        pltpu.sync_copy(src_hbm.at[pl.ds(wid * _EPW + ch * _C, _C)],
                        schunk.at[pl.ds(0, _C)])
        pltpu.sync_copy(dst_hbm.at[pl.ds(wid * _EPW + ch * _C, _C)],
                        dchunk.at[pl.ds(0, _C)])

        def edge_body(e, carry2):
            sv = schunk[pl.ds(e, 16)][0]
            dv = dchunk[pl.ds(e, 16)][0]
            o = (dv * _DIVM) >> _DIVS
            dl = dv - o * _NPW
            co = cntb[pl.ds(o * _CSTRIDE, 16)][0]
            bs[pl.ds(o * _BSTRIDE + co, 16)] = jnp.full((16,), sv, jnp.int32)
            bd[pl.ds(o * _BSTRIDE + co, 16)] = jnp.full((16,), dl, jnp.int32)

            @pl.when(co == _K - 1)
            def _():
                nf = nflb[pl.ds(o * _CSTRIDE, 16)][0]
                flush(o, nf, _K)
                nflb[pl.ds(o * _CSTRIDE, 16)] = jnp.full((16,), nf + 1, jnp.int32)

            cntb[pl.ds(o * _CSTRIDE, 16)] = jnp.full(
                (16,), jnp.where(co == _K - 1, 0, co + 1), jnp.int32)
            return carry2

        lax.fori_loop(0, _C, edge_body, 0)
        return carry

    lax.fori_loop(0, _NCHUNK, chunk_body, 0)

    def tail_body(o, carry):
        nf = nflb[pl.ds(o * _CSTRIDE, 16)][0]
        flush(o, nf, 0)
        nflb[pl.ds(o * _CSTRIDE, 16)] = jnp.full((16,), nf + 1, jnp.int32)
        return carry

    lax.fori_loop(0, _NW, tail_body, 0)

    pltpu.sync_copy(nflb, cnts_hbm.at[pl.ds(wid * _NW * _CSTRIDE, _NW * _CSTRIDE)])


_sc_bin = functools.partial(
    pl.kernel,
    out_type=(
        jax.ShapeDtypeStruct((_NW * _NW * _RCAP,), jnp.int32),
        jax.ShapeDtypeStruct((_NW * _NW * _RCAP,), jnp.int32),
        jax.ShapeDtypeStruct((_NW * _NW * _CSTRIDE,), jnp.int32),
    ),
    mesh=plsc.VectorSubcoreMesh(core_axis_name="c", subcore_axis_name="s"),
    scratch_types=[
        pltpu.VMEM((_C + 16,), jnp.int32),
        pltpu.VMEM((_C + 16,), jnp.int32),
        pltpu.VMEM((_NW * _BSTRIDE,), jnp.int32),
        pltpu.VMEM((_NW * _BSTRIDE,), jnp.int32),
        pltpu.VMEM((_NW * _CSTRIDE,), jnp.int32),
        pltpu.VMEM((_NW * _CSTRIDE,), jnp.int32),
        pltpu.VMEM((32,), jnp.int32),
        pltpu.SemaphoreType.DMA,
    ],
)(_sc_bin_body)


def _sc_drain_body(h_hbm, sidx_hbm, didx_hbm, cnts_hbm, out_hbm,
                   acc, cvm, sb0, sb1, db0, db1, rw0, rw1,
                   semi0, semi1, semg0, semg1):
    wid = _tile_id()
    lo = wid * _NPW

    neg = jnp.full((16,), -jnp.inf, dtype=jnp.float32)

    def init_body(r, carry):
        for c in range(_D // 16):
            acc[r, pl.ds(c * 16, 16)] = neg
        return carry

    lax.fori_loop(0, _NPW, init_body, 0)

    pltpu.sync_copy(cnts_hbm, cvm)

    def start_idx(base, sb, db, sem):
        pltpu.async_copy(sidx_hbm.at[pl.ds(base, _K)], sb, sem)
        pltpu.async_copy(didx_hbm.at[pl.ds(base, _K)], db.at[pl.ds(0, _K)], sem)

    def wait_idx(sb, db, sem):
        pltpu.make_async_copy(sidx_hbm.at[pl.ds(0, _K)], sb, sem).wait()
        pltpu.make_async_copy(didx_hbm.at[pl.ds(0, _K)], db.at[pl.ds(0, _K)], sem).wait()

    def start_gather(sb, rw, sem):
        pltpu.async_copy(h_hbm.at[sb], rw, sem)

    def wait_gather(rw, sem):
        pltpu.make_async_copy(h_hbm.at[pl.ds(0, _K)], rw, sem).wait()

    def accum(rw, db):
        @plsc.parallel_loop(0, _D // 16)
        def c_body(c):
            sl = pl.ds(c * 16, 16)

            def g_body(g, carry):
                g16 = g * 16
                dv = db[pl.ds(g16, 16)]
                for k in range(16):
                    d = dv[k]
                    acc[d, sl] = jnp.maximum(acc[d, sl], rw[g16 + k, sl])
                return carry

            lax.fori_loop(0, _K // 16, g_body, 0)

    def nfl_rbase(t):
        nfl = cvm[pl.ds(t * _NW * _CSTRIDE + wid * _CSTRIDE, 16)][0]
        return nfl, (t * _NW + wid) * _RCAP

    def base_of(t, b):
        nfl, rbase = nfl_rbase(t)
        return rbase + jnp.minimum(b, nfl - 1) * _K

    start_idx(base_of(0, 0), sb0, db0, semi0)
    start_idx(base_of(0, 1), sb1, db1, semi1)

    def src_body(t, carry):
        nfl, rbase = nfl_rbase(t)
        clamp = nfl - 1

        wait_idx(sb0, db0, semi0)
        start_gather(sb0, rw0, semg0)
        nit = nfl >> 1

        def it_body(k, carry2):
            wait_idx(sb1, db1, semi1)
            start_gather(sb1, rw1, semg1)
            wait_gather(rw0, semg0)
            accum(rw0, db0)
            start_idx(rbase + jnp.minimum(2 * k + 2, clamp) * _K, sb0, db0, semi0)
            wait_gather(rw1, semg1)
            accum(rw1, db1)
            start_idx(rbase + jnp.minimum(2 * k + 3, clamp) * _K, sb1, db1, semi1)
            wait_idx(sb0, db0, semi0)
            start_gather(sb0, rw0, semg0)
            return carry2

        lax.fori_loop(0, nit, it_body, 0)
        wait_gather(rw0, semg0)
        tn = jnp.minimum(t + 1, _NW - 1)

        @pl.when((nfl & 1) == 1)
        def _():
            accum(rw0, db0)

        wait_idx(sb1, db1, semi1)
        start_idx(base_of(tn, 0), sb0, db0, semi0)
        start_idx(base_of(tn, 1), sb1, db1, semi1)
        return carry

    lax.fori_loop(0, _NW, src_body, 0)
    wait_idx(sb0, db0, semi0)
    wait_idx(sb1, db1, semi1)

    pltpu.sync_copy(acc.at[pl.ds(0, _NPW)], out_hbm.at[pl.ds(lo, _NPW)])


_sc_drain = functools.partial(
    pl.kernel,
    out_type=jax.ShapeDtypeStruct((_NPAD, _D), jnp.float32),
    mesh=plsc.VectorSubcoreMesh(core_axis_name="c", subcore_axis_name="s"),
    scratch_types=[
        pltpu.VMEM((_NPW + 1, _D), jnp.float32),
        pltpu.VMEM((_NW * _NW * _CSTRIDE,), jnp.int32),
        pltpu.VMEM((_K,), jnp.int32),
        pltpu.VMEM((_K,), jnp.int32),
        pltpu.VMEM((_K + 16,), jnp.int32),
        pltpu.VMEM((_K + 16,), jnp.int32),
        pltpu.VMEM((_K, _D), jnp.float32),
        pltpu.VMEM((_K, _D), jnp.float32),
        pltpu.SemaphoreType.DMA,
        pltpu.SemaphoreType.DMA,
        pltpu.SemaphoreType.DMA,
        pltpu.SemaphoreType.DMA,
    ],
)(_sc_drain_body)


def _tc_body(h_ref, a_ref, wt_ref, b_ref, o_ref, *, relu):
    a = a_ref[...]
    x = h_ref[...] + jnp.where(a == -jnp.inf, 0.0, a)
    y = jnp.dot(x, wt_ref[...], preferred_element_type=jnp.float32) + b_ref[...]
    if relu:
        y = jnp.maximum(y, 0.0)
    o_ref[...] = y


def _tc_linear(h, agg, wt, b, relu):
    blk = 1000
    return pl.pallas_call(
        functools.partial(_tc_body, relu=relu),
        grid=(_N // blk,),
        in_specs=[
            pl.BlockSpec((blk, _D), lambda i: (i, 0)),
            pl.BlockSpec((blk, _D), lambda i: (i, 0)),
            pl.BlockSpec((_D, _D), lambda i: (0, 0)),
            pl.BlockSpec((1, _D), lambda i: (0, 0)),
        ],
        out_specs=pl.BlockSpec((blk, _D), lambda i: (i, 0)),
        out_shape=jax.ShapeDtypeStruct((_N, _D), jnp.float32),
    )(h, agg, wt, b.reshape(1, _D))


def kernel(h, edge_index, W1, b1, W2, b2):
    src = edge_index[0]
    dst = edge_index[1]
    sidx, didx, cnts = _sc_bin(src, dst)
    agg1 = _sc_drain(h, sidx, didx, cnts)
    h1 = _tc_linear(h, agg1[:_N], W1.T, b1, relu=True)
    agg2 = _sc_drain(h1, sidx, didx, cnts)
    return _tc_linear(h1, agg2[:_N], W2.T, b2, relu=False)


# speedup vs baseline: 1.1468x; 1.1468x over previous
import functools

import jax
import jax.numpy as jnp
from jax import lax
from jax.experimental import pallas as pl
from jax.experimental.pallas import tpu as pltpu
from jax.experimental.pallas import tpu_sc as plsc

_N = 10000
_E = 320000
_D = 128
_NW = 32
_NPW = 320
_NPAD = _NW * _NPW
_EPW = _E // _NW
_C = 2000
_NCHUNK = _EPW // _C
_K = 128
_BSTRIDE = _K + 16
_NBLK = _EPW // _K + 1
_RCAP = _NBLK * _K
_CSTRIDE = 16
_DIVM = 13108
_DIVS = 22


def _tile_id():
    return lax.axis_index("s") * 2 + lax.axis_index("c")


def _sc_bin_body(src_hbm, dst_hbm, sidx_hbm, didx_hbm, cnts_hbm,
                 schunk, dchunk, bs, bd, cntb, nflb, cbuf, sem):
    wid = _tile_id()

    zero16 = jnp.zeros((16,), dtype=jnp.int32)
    trash16 = jnp.full((16,), _NPW, dtype=jnp.int32)
    for o in range(_NW):
        cntb[pl.ds(o * _CSTRIDE, 16)] = zero16
        nflb[pl.ds(o * _CSTRIDE, 16)] = zero16
        for t in range(_BSTRIDE // 16):
            bs[pl.ds(o * _BSTRIDE + t * 16, 16)] = zero16
            bd[pl.ds(o * _BSTRIDE + t * 16, 16)] = trash16

    def flush(o, nf, valid):
        base = (wid * _NW) * _RCAP + o * _RCAP + nf * _K
        pltpu.sync_copy(bs.at[pl.ds(o * _BSTRIDE, _K)],
                        sidx_hbm.at[pl.ds(base, _K)])
        pltpu.sync_copy(bd.at[pl.ds(o * _BSTRIDE, _K)],
                        didx_hbm.at[pl.ds(base, _K)])
        del valid

    def chunk_body(ch, carry):
        pltpu.sync_copy(src_hbm.at[pl.ds(wid * _EPW + ch * _C, _C)],
                        schunk.at[pl.ds(0, _C)])
        pltpu.sync_copy(dst_hbm.at[pl.ds(wid * _EPW + ch * _C, _C)],
                        dchunk.at[pl.ds(0, _C)])

        def edge_body(e, carry2):
            sv = schunk[pl.ds(e, 16)][0]
            dv = dchunk[pl.ds(e, 16)][0]
            o = (dv * _DIVM) >> _DIVS
            dl = dv - o * _NPW
            co = cntb[pl.ds(o * _CSTRIDE, 16)][0]
            bs[pl.ds(o * _BSTRIDE + co, 16)] = jnp.full((16,), sv, jnp.int32)
            bd[pl.ds(o * _BSTRIDE + co, 16)] = jnp.full((16,), dl, jnp.int32)

            @pl.when(co == _K - 1)
            def _():
                nf = nflb[pl.ds(o * _CSTRIDE, 16)][0]
                flush(o, nf, _K)
                nflb[pl.ds(o * _CSTRIDE, 16)] = jnp.full((16,), nf + 1, jnp.int32)

            cntb[pl.ds(o * _CSTRIDE, 16)] = jnp.full(
                (16,), jnp.where(co == _K - 1, 0, co + 1), jnp.int32)
            return carry2

        lax.fori_loop(0, _C, edge_body, 0)
        return carry

    lax.fori_loop(0, _NCHUNK, chunk_body, 0)

    def tail_body(o, carry):
        nf = nflb[pl.ds(o * _CSTRIDE, 16)][0]
        flush(o, nf, 0)
        nflb[pl.ds(o * _CSTRIDE, 16)] = jnp.full((16,), nf + 1, jnp.int32)
        return carry

    lax.fori_loop(0, _NW, tail_body, 0)

    pltpu.sync_copy(nflb, cnts_hbm.at[pl.ds(wid * _NW * _CSTRIDE, _NW * _CSTRIDE)])


_sc_bin = functools.partial(
    pl.kernel,
    out_type=(
        jax.ShapeDtypeStruct((_NW * _NW * _RCAP,), jnp.int32),
        jax.ShapeDtypeStruct((_NW * _NW * _RCAP,), jnp.int32),
        jax.ShapeDtypeStruct((_NW * _NW * _CSTRIDE,), jnp.int32),
    ),
    mesh=plsc.VectorSubcoreMesh(core_axis_name="c", subcore_axis_name="s"),
    scratch_types=[
        pltpu.VMEM((_C + 16,), jnp.int32),
        pltpu.VMEM((_C + 16,), jnp.int32),
        pltpu.VMEM((_NW * _BSTRIDE,), jnp.int32),
        pltpu.VMEM((_NW * _BSTRIDE,), jnp.int32),
        pltpu.VMEM((_NW * _CSTRIDE,), jnp.int32),
        pltpu.VMEM((_NW * _CSTRIDE,), jnp.int32),
        pltpu.VMEM((32,), jnp.int32),
        pltpu.SemaphoreType.DMA,
    ],
)(_sc_bin_body)


def _sc_drain_body(h_hbm, sidx_hbm, didx_hbm, cnts_hbm, out_hbm,
                   acc, cvm, sb0, sb1, db0, db1, rw0, rw1,
                   semi0, semi1, semg0, semg1):
    wid = _tile_id()
    lo = wid * _NPW

    neg = jnp.full((16,), -jnp.inf, dtype=jnp.float32)

    def init_body(r, carry):
        for c in range(_D // 16):
            acc[r, pl.ds(c * 16, 16)] = neg
        return carry

    lax.fori_loop(0, _NPW, init_body, 0)

    pltpu.sync_copy(cnts_hbm, cvm)

    def start_idx(base, sb, db, sem):
        pltpu.async_copy(sidx_hbm.at[pl.ds(base, _K)], sb, sem)
        pltpu.async_copy(didx_hbm.at[pl.ds(base, _K)], db.at[pl.ds(0, _K)], sem)

    def wait_idx(sb, db, sem):
        pltpu.make_async_copy(sidx_hbm.at[pl.ds(0, _K)], sb, sem).wait()
        pltpu.make_async_copy(didx_hbm.at[pl.ds(0, _K)], db.at[pl.ds(0, _K)], sem).wait()

    def start_gather(sb, rw, sem):
        pltpu.async_copy(h_hbm.at[sb], rw, sem)

    def wait_gather(rw, sem):
        pltpu.make_async_copy(h_hbm.at[pl.ds(0, _K)], rw, sem).wait()

    def accum(rw, db):
        def g_body(g, carry):
            g16 = g * 16
            dv = db[pl.ds(g16, 16)]
            for k in range(16):
                d = dv[k]
                for c in range(_D // 16):
                    sl = pl.ds(c * 16, 16)
                    acc[d, sl] = jnp.maximum(acc[d, sl], rw[g16 + k, sl])
            return carry

        lax.fori_loop(0, _K // 16, g_body, 0)

    def nfl_rbase(t):
        nfl = cvm[pl.ds(t * _NW * _CSTRIDE + wid * _CSTRIDE, 16)][0]
        return nfl, (t * _NW + wid) * _RCAP

    def base_of(t, b):
        nfl, rbase = nfl_rbase(t)
        return rbase + jnp.minimum(b, nfl - 1) * _K

    start_idx(base_of(0, 0), sb0, db0, semi0)
    start_idx(base_of(0, 1), sb1, db1, semi1)

    def src_body(t, carry):
        nfl, rbase = nfl_rbase(t)
        clamp = nfl - 1

        wait_idx(sb0, db0, semi0)
        start_gather(sb0, rw0, semg0)
        nit = nfl >> 1

        def it_body(k, carry2):
            wait_idx(sb1, db1, semi1)
            start_gather(sb1, rw1, semg1)
            wait_gather(rw0, semg0)
            accum(rw0, db0)
            start_idx(rbase + jnp.minimum(2 * k + 2, clamp) * _K, sb0, db0, semi0)
            wait_gather(rw1, semg1)
            accum(rw1, db1)
            start_idx(rbase + jnp.minimum(2 * k + 3, clamp) * _K, sb1, db1, semi1)
            wait_idx(sb0, db0, semi0)
            start_gather(sb0, rw0, semg0)
            return carry2

        lax.fori_loop(0, nit, it_body, 0)
        wait_gather(rw0, semg0)
        tn = jnp.minimum(t + 1, _NW - 1)

        @pl.when((nfl & 1) == 1)
        def _():
            accum(rw0, db0)

        wait_idx(sb1, db1, semi1)
        start_idx(base_of(tn, 0), sb0, db0, semi0)
        start_idx(base_of(tn, 1), sb1, db1, semi1)
        return carry

    lax.fori_loop(0, _NW, src_body, 0)
    wait_idx(sb0, db0, semi0)
    wait_idx(sb1, db1, semi1)

    pltpu.sync_copy(acc.at[pl.ds(0, _NPW)], out_hbm.at[pl.ds(lo, _NPW)])


_sc_drain = functools.partial(
    pl.kernel,
    out_type=jax.ShapeDtypeStruct((_NPAD, _D), jnp.float32),
    mesh=plsc.VectorSubcoreMesh(core_axis_name="c", subcore_axis_name="s"),
    scratch_types=[
        pltpu.VMEM((_NPW + 1, _D), jnp.float32),
        pltpu.VMEM((_NW * _NW * _CSTRIDE,), jnp.int32),
        pltpu.VMEM((_K,), jnp.int32),
        pltpu.VMEM((_K,), jnp.int32),
        pltpu.VMEM((_K + 16,), jnp.int32),
        pltpu.VMEM((_K + 16,), jnp.int32),
        pltpu.VMEM((_K, _D), jnp.float32),
        pltpu.VMEM((_K, _D), jnp.float32),
        pltpu.SemaphoreType.DMA,
        pltpu.SemaphoreType.DMA,
        pltpu.SemaphoreType.DMA,
        pltpu.SemaphoreType.DMA,
    ],
)(_sc_drain_body)


def _tc_body(h_ref, a_ref, wt_ref, b_ref, o_ref, *, relu):
    a = a_ref[...]
    x = h_ref[...] + jnp.where(a == -jnp.inf, 0.0, a)
    y = jnp.dot(x, wt_ref[...], preferred_element_type=jnp.float32) + b_ref[...]
    if relu:
        y = jnp.maximum(y, 0.0)
    o_ref[...] = y


def _tc_linear(h, agg, wt, b, relu):
    blk = 1000
    return pl.pallas_call(
        functools.partial(_tc_body, relu=relu),
        grid=(_N // blk,),
        in_specs=[
            pl.BlockSpec((blk, _D), lambda i: (i, 0)),
            pl.BlockSpec((blk, _D), lambda i: (i, 0)),
            pl.BlockSpec((_D, _D), lambda i: (0, 0)),
            pl.BlockSpec((1, _D), lambda i: (0, 0)),
        ],
        out_specs=pl.BlockSpec((blk, _D), lambda i: (i, 0)),
        out_shape=jax.ShapeDtypeStruct((_N, _D), jnp.float32),
    )(h, agg, wt, b.reshape(1, _D))


def kernel(h, edge_index, W1, b1, W2, b2):
    src = edge_index[0]
    dst = edge_index[1]
    sidx, didx, cnts = _sc_bin(src, dst)
    agg1 = _sc_drain(h, sidx, didx, cnts)
    h1 = _tc_linear(h, agg1[:_N], W1.T, b1, relu=True)
    agg2 = _sc_drain(h1, sidx, didx, cnts)
    return _tc_linear(h1, agg2[:_N], W2.T, b2, relu=False)


# speedup vs baseline: 1.1861x; 1.0342x over previous
import functools

import jax
import jax.numpy as jnp
from jax import lax
from jax.experimental import pallas as pl
from jax.experimental.pallas import tpu as pltpu
from jax.experimental.pallas import tpu_sc as plsc

_N = 10000
_E = 320000
_D = 128
_NW = 32
_NPW = 320
_NPAD = _NW * _NPW
_EPW = _E // _NW
_C = 2000
_NCHUNK = _EPW // _C
_K = 128
_BSTRIDE = _K + 16
_NBLK = _EPW // _K + 1
_RCAP = _NBLK * _K
_CSTRIDE = 16
_DIVM = 13108
_DIVS = 22


def _tile_id():
    return lax.axis_index("s") * 2 + lax.axis_index("c")


def _sc_bin_body(src_hbm, dst_hbm, sidx_hbm, didx_hbm, cnts_hbm,
                 schunk, dchunk, bs, bd, cntb, nflb, cbuf, sem):
    wid = _tile_id()

    zero16 = jnp.zeros((16,), dtype=jnp.int32)
    trash16 = jnp.full((16,), _NPW, dtype=jnp.int32)
    for o in range(_NW):
        cntb[pl.ds(o * _CSTRIDE, 16)] = zero16
        nflb[pl.ds(o * _CSTRIDE, 16)] = zero16
        for t in range(_BSTRIDE // 16):
            bs[pl.ds(o * _BSTRIDE + t * 16, 16)] = zero16
            bd[pl.ds(o * _BSTRIDE + t * 16, 16)] = trash16

    def flush(o, nf, valid):
        base = (wid * _NW) * _RCAP + o * _RCAP + nf * _K
        pltpu.sync_copy(bs.at[pl.ds(o * _BSTRIDE, _K)],
                        sidx_hbm.at[pl.ds(base, _K)])
        pltpu.sync_copy(bd.at[pl.ds(o * _BSTRIDE, _K)],
                        didx_hbm.at[pl.ds(base, _K)])
        del valid

    def chunk_body(ch, carry):
        pltpu.sync_copy(src_hbm.at[pl.ds(wid * _EPW + ch * _C, _C)],
                        schunk.at[pl.ds(0, _C)])
        pltpu.sync_copy(dst_hbm.at[pl.ds(wid * _EPW + ch * _C, _C)],
                        dchunk.at[pl.ds(0, _C)])

        def sub_body(i, carry2):
            svv = schunk[pl.ds(i * 16, 16)]
            dvv = dchunk[pl.ds(i * 16, 16)]
            ovv = (dvv * _DIVM) >> _DIVS
            dlv = dvv - ovv * _NPW
            for k in range(16):
                sv = svv[k]
                o = ovv[k]
                dl = dlv[k]
                co = cntb[pl.ds(o * _CSTRIDE, 16)][0]
                bs[pl.ds(o * _BSTRIDE + co, 16)] = jnp.full((16,), sv, jnp.int32)
                bd[pl.ds(o * _BSTRIDE + co, 16)] = jnp.full((16,), dl, jnp.int32)

                @pl.when(co == _K - 1)
                def _():
                    nf = nflb[pl.ds(o * _CSTRIDE, 16)][0]
                    flush(o, nf, _K)
                    nflb[pl.ds(o * _CSTRIDE, 16)] = jnp.full((16,), nf + 1, jnp.int32)

                cntb[pl.ds(o * _CSTRIDE, 16)] = jnp.full(
                    (16,), jnp.where(co == _K - 1, 0, co + 1), jnp.int32)
            return carry2

        lax.fori_loop(0, _C // 16, sub_body, 0)
        return carry

    lax.fori_loop(0, _NCHUNK, chunk_body, 0)

    def tail_body(o, carry):
        nf = nflb[pl.ds(o * _CSTRIDE, 16)][0]
        flush(o, nf, 0)
        nflb[pl.ds(o * _CSTRIDE, 16)] = jnp.full((16,), nf + 1, jnp.int32)
        return carry

    lax.fori_loop(0, _NW, tail_body, 0)

    pltpu.sync_copy(nflb, cnts_hbm.at[pl.ds(wid * _NW * _CSTRIDE, _NW * _CSTRIDE)])


_sc_bin = functools.partial(
    pl.kernel,
    out_type=(
        jax.ShapeDtypeStruct((_NW * _NW * _RCAP,), jnp.int32),
        jax.ShapeDtypeStruct((_NW * _NW * _RCAP,), jnp.int32),
        jax.ShapeDtypeStruct((_NW * _NW * _CSTRIDE,), jnp.int32),
    ),
    mesh=plsc.VectorSubcoreMesh(core_axis_name="c", subcore_axis_name="s"),
    scratch_types=[
        pltpu.VMEM((_C + 16,), jnp.int32),
        pltpu.VMEM((_C + 16,), jnp.int32),
        pltpu.VMEM((_NW * _BSTRIDE,), jnp.int32),
        pltpu.VMEM((_NW * _BSTRIDE,), jnp.int32),
        pltpu.VMEM((_NW * _CSTRIDE,), jnp.int32),
        pltpu.VMEM((_NW * _CSTRIDE,), jnp.int32),
        pltpu.VMEM((32,), jnp.int32),
        pltpu.SemaphoreType.DMA,
    ],
)(_sc_bin_body)


def _sc_drain_body(h_hbm, sidx_hbm, didx_hbm, cnts_hbm, out_hbm,
                   acc, cvm, sb0, sb1, db0, db1, rw0, rw1,
                   semi0, semi1, semg0, semg1):
    wid = _tile_id()
    lo = wid * _NPW

    neg = jnp.full((16,), -jnp.inf, dtype=jnp.float32)

    def init_body(r, carry):
        for c in range(_D // 16):
            acc[r, pl.ds(c * 16, 16)] = neg
        return carry

    lax.fori_loop(0, _NPW, init_body, 0)

    pltpu.sync_copy(cnts_hbm, cvm)

    def start_idx(base, sb, db, sem):
        pltpu.async_copy(sidx_hbm.at[pl.ds(base, _K)], sb, sem)
        pltpu.async_copy(didx_hbm.at[pl.ds(base, _K)], db.at[pl.ds(0, _K)], sem)

    def wait_idx(sb, db, sem):
        pltpu.make_async_copy(sidx_hbm.at[pl.ds(0, _K)], sb, sem).wait()
        pltpu.make_async_copy(didx_hbm.at[pl.ds(0, _K)], db.at[pl.ds(0, _K)], sem).wait()

    def start_gather(sb, rw, sem):
        pltpu.async_copy(h_hbm.at[sb], rw, sem)

    def wait_gather(rw, sem):
        pltpu.make_async_copy(h_hbm.at[pl.ds(0, _K)], rw, sem).wait()

    def accum(rw, db):
        def g_body(g, carry):
            g16 = g * 16
            dv = db[pl.ds(g16, 16)]
            for k in range(16):
                d = dv[k]
                for c in range(_D // 16):
                    sl = pl.ds(c * 16, 16)
                    acc[d, sl] = jnp.maximum(acc[d, sl], rw[g16 + k, sl])
            return carry

        lax.fori_loop(0, _K // 16, g_body, 0)

    def nfl_rbase(t):
        nfl = cvm[pl.ds(t * _NW * _CSTRIDE + wid * _CSTRIDE, 16)][0]
        return nfl, (t * _NW + wid) * _RCAP

    def base_of(t, b):
        nfl, rbase = nfl_rbase(t)
        return rbase + jnp.minimum(b, nfl - 1) * _K

    start_idx(base_of(0, 0), sb0, db0, semi0)
    start_idx(base_of(0, 1), sb1, db1, semi1)

    def src_body(t, carry):
        nfl, rbase = nfl_rbase(t)
        clamp = nfl - 1

        wait_idx(sb0, db0, semi0)
        start_gather(sb0, rw0, semg0)
        nit = nfl >> 1

        def it_body(k, carry2):
            wait_idx(sb1, db1, semi1)
            start_gather(sb1, rw1, semg1)
            wait_gather(rw0, semg0)
            accum(rw0, db0)
            start_idx(rbase + jnp.minimum(2 * k + 2, clamp) * _K, sb0, db0, semi0)
            wait_gather(rw1, semg1)
            accum(rw1, db1)
            start_idx(rbase + jnp.minimum(2 * k + 3, clamp) * _K, sb1, db1, semi1)
            wait_idx(sb0, db0, semi0)
            start_gather(sb0, rw0, semg0)
            return carry2

        lax.fori_loop(0, nit, it_body, 0)
        wait_gather(rw0, semg0)
        tn = jnp.minimum(t + 1, _NW - 1)

        @pl.when((nfl & 1) == 1)
        def _():
            accum(rw0, db0)

        wait_idx(sb1, db1, semi1)
        start_idx(base_of(tn, 0), sb0, db0, semi0)
        start_idx(base_of(tn, 1), sb1, db1, semi1)
        return carry

    lax.fori_loop(0, _NW, src_body, 0)
    wait_idx(sb0, db0, semi0)
    wait_idx(sb1, db1, semi1)

    pltpu.sync_copy(acc.at[pl.ds(0, _NPW)], out_hbm.at[pl.ds(lo, _NPW)])


_sc_drain = functools.partial(
    pl.kernel,
    out_type=jax.ShapeDtypeStruct((_NPAD, _D), jnp.float32),
    mesh=plsc.VectorSubcoreMesh(core_axis_name="c", subcore_axis_name="s"),
    scratch_types=[
        pltpu.VMEM((_NPW + 1, _D), jnp.float32),
        pltpu.VMEM((_NW * _NW * _CSTRIDE,), jnp.int32),
        pltpu.VMEM((_K,), jnp.int32),
        pltpu.VMEM((_K,), jnp.int32),
        pltpu.VMEM((_K + 16,), jnp.int32),
        pltpu.VMEM((_K + 16,), jnp.int32),
        pltpu.VMEM((_K, _D), jnp.float32),
        pltpu.VMEM((_K, _D), jnp.float32),
        pltpu.SemaphoreType.DMA,
        pltpu.SemaphoreType.DMA,
        pltpu.SemaphoreType.DMA,
        pltpu.SemaphoreType.DMA,
    ],
)(_sc_drain_body)


def _tc_body(h_ref, a_ref, wt_ref, b_ref, o_ref, *, relu):
    a = a_ref[...]
    x = h_ref[...] + jnp.where(a == -jnp.inf, 0.0, a)
    y = jnp.dot(x, wt_ref[...], preferred_element_type=jnp.float32) + b_ref[...]
    if relu:
        y = jnp.maximum(y, 0.0)
    o_ref[...] = y


def _tc_linear(h, agg, wt, b, relu):
    blk = 1000
    return pl.pallas_call(
        functools.partial(_tc_body, relu=relu),
        grid=(_N // blk,),
        in_specs=[
            pl.BlockSpec((blk, _D), lambda i: (i, 0)),
            pl.BlockSpec((blk, _D), lambda i: (i, 0)),
            pl.BlockSpec((_D, _D), lambda i: (0, 0)),
            pl.BlockSpec((1, _D), lambda i: (0, 0)),
        ],
        out_specs=pl.BlockSpec((blk, _D), lambda i: (i, 0)),
        out_shape=jax.ShapeDtypeStruct((_N, _D), jnp.float32),
    )(h, agg, wt, b.reshape(1, _D))


def kernel(h, edge_index, W1, b1, W2, b2):
    src = edge_index[0]
    dst = edge_index[1]
    sidx, didx, cnts = _sc_bin(src, dst)
    agg1 = _sc_drain(h, sidx, didx, cnts)
    h1 = _tc_linear(h, agg1[:_N], W1.T, b1, relu=True)
    agg2 = _sc_drain(h1, sidx, didx, cnts)
    return _tc_linear(h1, agg2[:_N], W2.T, b2, relu=False)


# speedup vs baseline: 1.3211x; 1.1139x over previous
import functools

import jax
import jax.numpy as jnp
from jax import lax
from jax.experimental import pallas as pl
from jax.experimental.pallas import tpu as pltpu
from jax.experimental.pallas import tpu_sc as plsc

_N = 10000
_E = 320000
_D = 128
_NW = 32
_NPW = 320
_NPAD = _NW * _NPW
_EPW = _E // _NW
_C = 2000
_NCHUNK = _EPW // _C
_K = 128
_BSTRIDE = _K + 16
_NBLK = _EPW // _K + 1
_RCAP = _NBLK * _K
_CSTRIDE = 16
_DIVM = 13108
_DIVS = 22


def _tile_id():
    return lax.axis_index("s") * 2 + lax.axis_index("c")


def _sc_bin_body(src_hbm, dst_hbm, sidx_hbm, didx_hbm, cnts_hbm,
                 schunk, dchunk, bs, bd, cntb, nflb, stg_s, stg_d, rbuf, semf):
    wid = _tile_id()

    zero16 = jnp.zeros((16,), dtype=jnp.int32)
    trash16 = jnp.full((16,), _NPW, dtype=jnp.int32)
    rbuf[pl.ds(0, 16)] = zero16
    for o in range(_NW):
        cntb[pl.ds(o * _CSTRIDE, 16)] = zero16
        nflb[pl.ds(o * _CSTRIDE, 16)] = zero16
        for t in range(_BSTRIDE // 16):
            bs[pl.ds(o * _BSTRIDE + t * 16, 16)] = zero16
            bd[pl.ds(o * _BSTRIDE + t * 16, 16)] = trash16

    def wait_flush_pair():
        pltpu.make_async_copy(sidx_hbm.at[pl.ds(0, _K)],
                              stg_s.at[pl.ds(0, _K)], semf).wait()
        pltpu.make_async_copy(didx_hbm.at[pl.ds(0, _K)],
                              stg_d.at[pl.ds(0, _K)], semf).wait()

    def flush(o, nf, valid):
        base = (wid * _NW) * _RCAP + o * _RCAP + nf * _K
        fc = rbuf[pl.ds(0, 16)][0]
        slot = (fc & 3) * _K

        @pl.when(fc >= 4)
        def _():
            wait_flush_pair()

        for t in range(_K // 16):
            stg_s[pl.ds(slot + t * 16, 16)] = bs[pl.ds(o * _BSTRIDE + t * 16, 16)]
            stg_d[pl.ds(slot + t * 16, 16)] = bd[pl.ds(o * _BSTRIDE + t * 16, 16)]
        pltpu.async_copy(stg_s.at[pl.ds(slot, _K)],
                         sidx_hbm.at[pl.ds(base, _K)], semf)
        pltpu.async_copy(stg_d.at[pl.ds(slot, _K)],
                         didx_hbm.at[pl.ds(base, _K)], semf)
        rbuf[pl.ds(0, 16)] = jnp.full((16,), fc + 1, jnp.int32)
        del valid

    def chunk_body(ch, carry):
        pltpu.sync_copy(src_hbm.at[pl.ds(wid * _EPW + ch * _C, _C)],
                        schunk.at[pl.ds(0, _C)])
        pltpu.sync_copy(dst_hbm.at[pl.ds(wid * _EPW + ch * _C, _C)],
                        dchunk.at[pl.ds(0, _C)])

        def sub_body(i, carry2):
            svv = schunk[pl.ds(i * 16, 16)]
            dvv = dchunk[pl.ds(i * 16, 16)]
            ovv = (dvv * _DIVM) >> _DIVS
            dlv = dvv - ovv * _NPW
            for k in range(16):
                sv = svv[k]
                o = ovv[k]
                dl = dlv[k]
                co = cntb[pl.ds(o * _CSTRIDE, 16)][0]
                bs[pl.ds(o * _BSTRIDE + co, 16)] = jnp.full((16,), sv, jnp.int32)
                bd[pl.ds(o * _BSTRIDE + co, 16)] = jnp.full((16,), dl, jnp.int32)

                @pl.when(co == _K - 1)
                def _():
                    nf = nflb[pl.ds(o * _CSTRIDE, 16)][0]
                    flush(o, nf, _K)
                    nflb[pl.ds(o * _CSTRIDE, 16)] = jnp.full((16,), nf + 1, jnp.int32)

                cntb[pl.ds(o * _CSTRIDE, 16)] = jnp.full(
                    (16,), jnp.where(co == _K - 1, 0, co + 1), jnp.int32)
            return carry2

        lax.fori_loop(0, _C // 16, sub_body, 0)
        return carry

    lax.fori_loop(0, _NCHUNK, chunk_body, 0)

    def tail_body(o, carry):
        nf = nflb[pl.ds(o * _CSTRIDE, 16)][0]
        flush(o, nf, 0)
        nflb[pl.ds(o * _CSTRIDE, 16)] = jnp.full((16,), nf + 1, jnp.int32)
        return carry

    lax.fori_loop(0, _NW, tail_body, 0)

    fc = rbuf[pl.ds(0, 16)][0]

    def drain_body(j, carry):
        wait_flush_pair()
        return carry

    lax.fori_loop(0, jnp.minimum(fc, 4), drain_body, 0)

    pltpu.sync_copy(nflb, cnts_hbm.at[pl.ds(wid * _NW * _CSTRIDE, _NW * _CSTRIDE)])


_sc_bin = functools.partial(
    pl.kernel,
    out_type=(
        jax.ShapeDtypeStruct((_NW * _NW * _RCAP,), jnp.int32),
        jax.ShapeDtypeStruct((_NW * _NW * _RCAP,), jnp.int32),
        jax.ShapeDtypeStruct((_NW * _NW * _CSTRIDE,), jnp.int32),
    ),
    mesh=plsc.VectorSubcoreMesh(core_axis_name="c", subcore_axis_name="s"),
    scratch_types=[
        pltpu.VMEM((_C + 16,), jnp.int32),
        pltpu.VMEM((_C + 16,), jnp.int32),
        pltpu.VMEM((_NW * _BSTRIDE,), jnp.int32),
        pltpu.VMEM((_NW * _BSTRIDE,), jnp.int32),
        pltpu.VMEM((_NW * _CSTRIDE,), jnp.int32),
        pltpu.VMEM((_NW * _CSTRIDE,), jnp.int32),
        pltpu.VMEM((4 * _K,), jnp.int32),
        pltpu.VMEM((4 * _K,), jnp.int32),
        pltpu.VMEM((16,), jnp.int32),
        pltpu.SemaphoreType.DMA,
    ],
)(_sc_bin_body)


def _sc_drain_body(h_hbm, sidx_hbm, didx_hbm, cnts_hbm, out_hbm,
                   acc, cvm, sb0, sb1, db0, db1, rw0, rw1,
                   semi0, semi1, semg0, semg1):
    wid = _tile_id()
    lo = wid * _NPW

    neg = jnp.full((16,), -jnp.inf, dtype=jnp.float32)

    def init_body(r, carry):
        for c in range(_D // 16):
            acc[r, pl.ds(c * 16, 16)] = neg
        return carry

    lax.fori_loop(0, _NPW, init_body, 0)

    pltpu.sync_copy(cnts_hbm, cvm)

    def start_idx(base, sb, db, sem):
        pltpu.async_copy(sidx_hbm.at[pl.ds(base, _K)], sb, sem)
        pltpu.async_copy(didx_hbm.at[pl.ds(base, _K)], db.at[pl.ds(0, _K)], sem)

    def wait_idx(sb, db, sem):
        pltpu.make_async_copy(sidx_hbm.at[pl.ds(0, _K)], sb, sem).wait()
        pltpu.make_async_copy(didx_hbm.at[pl.ds(0, _K)], db.at[pl.ds(0, _K)], sem).wait()

    def start_gather(sb, rw, sem):
        pltpu.async_copy(h_hbm.at[sb], rw, sem)

    def wait_gather(rw, sem):
        pltpu.make_async_copy(h_hbm.at[pl.ds(0, _K)], rw, sem).wait()

    def accum(rw, db):
        def g_body(g, carry):
            g16 = g * 16
            dv = db[pl.ds(g16, 16)]
            for k in range(16):
                d = dv[k]
                for c in range(_D // 16):
                    sl = pl.ds(c * 16, 16)
                    acc[d, sl] = jnp.maximum(acc[d, sl], rw[g16 + k, sl])
            return carry

        lax.fori_loop(0, _K // 16, g_body, 0)

    def nfl_rbase(t):
        nfl = cvm[pl.ds(t * _NW * _CSTRIDE + wid * _CSTRIDE, 16)][0]
        return nfl, (t * _NW + wid) * _RCAP

    def base_of(t, b):
        nfl, rbase = nfl_rbase(t)
        return rbase + jnp.minimum(b, nfl - 1) * _K

    start_idx(base_of(0, 0), sb0, db0, semi0)
    start_idx(base_of(0, 1), sb1, db1, semi1)

    def src_body(t, carry):
        nfl, rbase = nfl_rbase(t)
        clamp = nfl - 1

        wait_idx(sb0, db0, semi0)
        start_gather(sb0, rw0, semg0)
        nit = nfl >> 1

        def it_body(k, carry2):
            wait_idx(sb1, db1, semi1)
            start_gather(sb1, rw1, semg1)
            wait_gather(rw0, semg0)
            accum(rw0, db0)
            start_idx(rbase + jnp.minimum(2 * k + 2, clamp) * _K, sb0, db0, semi0)
            wait_gather(rw1, semg1)
            accum(rw1, db1)
            start_idx(rbase + jnp.minimum(2 * k + 3, clamp) * _K, sb1, db1, semi1)
            wait_idx(sb0, db0, semi0)
            start_gather(sb0, rw0, semg0)
            return carry2

        lax.fori_loop(0, nit, it_body, 0)
        wait_gather(rw0, semg0)
        tn = jnp.minimum(t + 1, _NW - 1)

        @pl.when((nfl & 1) == 1)
        def _():
            accum(rw0, db0)

        wait_idx(sb1, db1, semi1)
        start_idx(base_of(tn, 0), sb0, db0, semi0)
        start_idx(base_of(tn, 1), sb1, db1, semi1)
        return carry

    lax.fori_loop(0, _NW, src_body, 0)
    wait_idx(sb0, db0, semi0)
    wait_idx(sb1, db1, semi1)

    pltpu.sync_copy(acc.at[pl.ds(0, _NPW)], out_hbm.at[pl.ds(lo, _NPW)])


_sc_drain = functools.partial(
    pl.kernel,
    out_type=jax.ShapeDtypeStruct((_NPAD, _D), jnp.float32),
    mesh=plsc.VectorSubcoreMesh(core_axis_name="c", subcore_axis_name="s"),
    scratch_types=[
        pltpu.VMEM((_NPW + 1, _D), jnp.float32),
        pltpu.VMEM((_NW * _NW * _CSTRIDE,), jnp.int32),
        pltpu.VMEM((_K,), jnp.int32),
        pltpu.VMEM((_K,), jnp.int32),
        pltpu.VMEM((_K + 16,), jnp.int32),
        pltpu.VMEM((_K + 16,), jnp.int32),
        pltpu.VMEM((_K, _D), jnp.float32),
        pltpu.VMEM((_K, _D), jnp.float32),
        pltpu.SemaphoreType.DMA,
        pltpu.SemaphoreType.DMA,
        pltpu.SemaphoreType.DMA,
        pltpu.SemaphoreType.DMA,
    ],
)(_sc_drain_body)


def _tc_body(h_ref, a_ref, wt_ref, b_ref, o_ref, *, relu):
    a = a_ref[...]
    x = h_ref[...] + jnp.where(a == -jnp.inf, 0.0, a)
    y = jnp.dot(x, wt_ref[...], preferred_element_type=jnp.float32) + b_ref[...]
    if relu:
        y = jnp.maximum(y, 0.0)
    o_ref[...] = y


def _tc_linear(h, agg, wt, b, relu):
    blk = 1000
    return pl.pallas_call(
        functools.partial(_tc_body, relu=relu),
        grid=(_N // blk,),
        in_specs=[
            pl.BlockSpec((blk, _D), lambda i: (i, 0)),
            pl.BlockSpec((blk, _D), lambda i: (i, 0)),
            pl.BlockSpec((_D, _D), lambda i: (0, 0)),
            pl.BlockSpec((1, _D), lambda i: (0, 0)),
        ],
        out_specs=pl.BlockSpec((blk, _D), lambda i: (i, 0)),
        out_shape=jax.ShapeDtypeStruct((_N, _D), jnp.float32),
    )(h, agg, wt, b.reshape(1, _D))


def kernel(h, edge_index, W1, b1, W2, b2):
    src = edge_index[0]
    dst = edge_index[1]
    sidx, didx, cnts = _sc_bin(src, dst)
    agg1 = _sc_drain(h, sidx, didx, cnts)
    h1 = _tc_linear(h, agg1[:_N], W1.T, b1, relu=True)
    agg2 = _sc_drain(h1, sidx, didx, cnts)
    return _tc_linear(h1, agg2[:_N], W2.T, b2, relu=False)


# speedup vs baseline: 1.4275x; 1.0805x over previous
import functools

import jax
import jax.numpy as jnp
from jax import lax
from jax.experimental import pallas as pl
from jax.experimental.pallas import tpu as pltpu
from jax.experimental.pallas import tpu_sc as plsc

_N = 10000
_E = 320000
_D = 128
_NW = 32
_NPW = 320
_NPAD = _NW * _NPW
_EPW = _E // _NW
_C = 2000
_NCHUNK = _EPW // _C
_K = 128
_BSTRIDE = _K + 16
_NBLK = _EPW // _K + 1
_RCAP = _NBLK * _K
_CSTRIDE = 16
_DIVM = 13108
_DIVS = 22


def _tile_id():
    return lax.axis_index("s") * 2 + lax.axis_index("c")


def _sc_bin_body(src_hbm, dst_hbm, sidx_hbm, didx_hbm, cnts_hbm, fills_hbm,
                 schunk, dchunk, bs, bd, cntb, nflb, fillb, stg_s, stg_d, rbuf, semf):
    wid = _tile_id()

    zero16 = jnp.zeros((16,), dtype=jnp.int32)
    trash16 = jnp.full((16,), _NPW, dtype=jnp.int32)
    rbuf[pl.ds(0, 16)] = zero16
    for o in range(_NW):
        cntb[pl.ds(o * _CSTRIDE, 16)] = zero16
        nflb[pl.ds(o * _CSTRIDE, 16)] = zero16
        for t in range(_BSTRIDE // 16):
            bs[pl.ds(o * _BSTRIDE + t * 16, 16)] = zero16
            bd[pl.ds(o * _BSTRIDE + t * 16, 16)] = trash16

    def wait_flush_pair():
        pltpu.make_async_copy(sidx_hbm.at[pl.ds(0, _K)],
                              stg_s.at[pl.ds(0, _K)], semf).wait()
        pltpu.make_async_copy(didx_hbm.at[pl.ds(0, _K)],
                              stg_d.at[pl.ds(0, _K)], semf).wait()

    def flush(o, nf, valid):
        base = (wid * _NW) * _RCAP + o * _RCAP + nf * _K
        fc = rbuf[pl.ds(0, 16)][0]
        slot = (fc & 3) * _K

        @pl.when(fc >= 4)
        def _():
            wait_flush_pair()

        for t in range(_K // 16):
            stg_s[pl.ds(slot + t * 16, 16)] = bs[pl.ds(o * _BSTRIDE + t * 16, 16)]
            stg_d[pl.ds(slot + t * 16, 16)] = bd[pl.ds(o * _BSTRIDE + t * 16, 16)]
        pltpu.async_copy(stg_s.at[pl.ds(slot, _K)],
                         sidx_hbm.at[pl.ds(base, _K)], semf)
        pltpu.async_copy(stg_d.at[pl.ds(slot, _K)],
                         didx_hbm.at[pl.ds(base, _K)], semf)
        rbuf[pl.ds(0, 16)] = jnp.full((16,), fc + 1, jnp.int32)
        del valid

    def chunk_body(ch, carry):
        pltpu.sync_copy(src_hbm.at[pl.ds(wid * _EPW + ch * _C, _C)],
                        schunk.at[pl.ds(0, _C)])
        pltpu.sync_copy(dst_hbm.at[pl.ds(wid * _EPW + ch * _C, _C)],
                        dchunk.at[pl.ds(0, _C)])

        def sub_body(i, carry2):
            svv = schunk[pl.ds(i * 16, 16)]
            dvv = dchunk[pl.ds(i * 16, 16)]
            ovv = (dvv * _DIVM) >> _DIVS
            dlv = dvv - ovv * _NPW
            for k in range(16):
                sv = svv[k]
                o = ovv[k]
                dl = dlv[k]
                co = cntb[pl.ds(o * _CSTRIDE, 16)][0]
                bs[pl.ds(o * _BSTRIDE + co, 16)] = jnp.full((16,), sv, jnp.int32)
                bd[pl.ds(o * _BSTRIDE + co, 16)] = jnp.full((16,), dl, jnp.int32)

                @pl.when(co == _K - 1)
                def _():
                    nf = nflb[pl.ds(o * _CSTRIDE, 16)][0]
                    flush(o, nf, _K)
                    nflb[pl.ds(o * _CSTRIDE, 16)] = jnp.full((16,), nf + 1, jnp.int32)

                cntb[pl.ds(o * _CSTRIDE, 16)] = jnp.full(
                    (16,), jnp.where(co == _K - 1, 0, co + 1), jnp.int32)
            return carry2

        lax.fori_loop(0, _C // 16, sub_body, 0)
        return carry

    lax.fori_loop(0, _NCHUNK, chunk_body, 0)

    def tail_body(o, carry):
        nf = nflb[pl.ds(o * _CSTRIDE, 16)][0]
        co = cntb[pl.ds(o * _CSTRIDE, 16)][0]
        flush(o, nf, 0)
        nflb[pl.ds(o * _CSTRIDE, 16)] = jnp.full((16,), nf + 1, jnp.int32)
        fillb[pl.ds(o * _CSTRIDE, 16)] = jnp.full((16,), co, jnp.int32)
        return carry

    lax.fori_loop(0, _NW, tail_body, 0)

    fc = rbuf[pl.ds(0, 16)][0]

    def drain_body(j, carry):
        wait_flush_pair()
        return carry

    lax.fori_loop(0, jnp.minimum(fc, 4), drain_body, 0)

    pltpu.sync_copy(nflb, cnts_hbm.at[pl.ds(wid * _NW * _CSTRIDE, _NW * _CSTRIDE)])
    pltpu.sync_copy(fillb, fills_hbm.at[pl.ds(wid * _NW * _CSTRIDE, _NW * _CSTRIDE)])


_sc_bin = functools.partial(
    pl.kernel,
    out_type=(
        jax.ShapeDtypeStruct((_NW * _NW * _RCAP,), jnp.int32),
        jax.ShapeDtypeStruct((_NW * _NW * _RCAP,), jnp.int32),
        jax.ShapeDtypeStruct((_NW * _NW * _CSTRIDE,), jnp.int32),
        jax.ShapeDtypeStruct((_NW * _NW * _CSTRIDE,), jnp.int32),
    ),
    mesh=plsc.VectorSubcoreMesh(core_axis_name="c", subcore_axis_name="s"),
    scratch_types=[
        pltpu.VMEM((_C + 16,), jnp.int32),
        pltpu.VMEM((_C + 16,), jnp.int32),
        pltpu.VMEM((_NW * _BSTRIDE,), jnp.int32),
        pltpu.VMEM((_NW * _BSTRIDE,), jnp.int32),
        pltpu.VMEM((_NW * _CSTRIDE,), jnp.int32),
        pltpu.VMEM((_NW * _CSTRIDE,), jnp.int32),
        pltpu.VMEM((_NW * _CSTRIDE,), jnp.int32),
        pltpu.VMEM((4 * _K,), jnp.int32),
        pltpu.VMEM((4 * _K,), jnp.int32),
        pltpu.VMEM((16,), jnp.int32),
        pltpu.SemaphoreType.DMA,
    ],
)(_sc_bin_body)


def _sc_drain_body(h_hbm, sidx_hbm, didx_hbm, cnts_hbm, fills_hbm, out_hbm,
                   acc, cvm, fvm, sb0, sb1, db0, db1, rw0, rw1,
                   semi0, semi1, semg0, semg1):
    wid = _tile_id()
    lo = wid * _NPW

    neg = jnp.full((16,), -jnp.inf, dtype=jnp.float32)

    def init_body(r, carry):
        for c in range(_D // 16):
            acc[r, pl.ds(c * 16, 16)] = neg
        return carry

    lax.fori_loop(0, _NPW, init_body, 0)

    pltpu.sync_copy(cnts_hbm, cvm)
    pltpu.sync_copy(fills_hbm, fvm)

    def start_idx(base, sb, db, sem):
        pltpu.async_copy(sidx_hbm.at[pl.ds(base, _K)], sb, sem)
        pltpu.async_copy(didx_hbm.at[pl.ds(base, _K)], db.at[pl.ds(0, _K)], sem)

    def wait_idx(sb, db, sem):
        pltpu.make_async_copy(sidx_hbm.at[pl.ds(0, _K)], sb, sem).wait()
        pltpu.make_async_copy(didx_hbm.at[pl.ds(0, _K)], db.at[pl.ds(0, _K)], sem).wait()

    def start_gather(sb, rw, sem):
        pltpu.async_copy(h_hbm.at[sb], rw, sem)

    def wait_gather(rw, sem):
        pltpu.make_async_copy(h_hbm.at[pl.ds(0, _K)], rw, sem).wait()

    def accum(rw, db, ng):
        def g_body(g, carry):
            g16 = g * 16
            dv = db[pl.ds(g16, 16)]
            for k in range(16):
                d = dv[k]
                for c in range(_D // 16):
                    sl = pl.ds(c * 16, 16)
                    acc[d, sl] = jnp.maximum(acc[d, sl], rw[g16 + k, sl])
            return carry

        lax.fori_loop(0, ng, g_body, 0)

    def nfl_rbase(t):
        nfl = cvm[pl.ds(t * _NW * _CSTRIDE + wid * _CSTRIDE, 16)][0]
        return nfl, (t * _NW + wid) * _RCAP

    def base_of(t, b):
        nfl, rbase = nfl_rbase(t)
        return rbase + jnp.minimum(b, nfl - 1) * _K

    start_idx(base_of(0, 0), sb0, db0, semi0)
    start_idx(base_of(0, 1), sb1, db1, semi1)

    def src_body(t, carry):
        nfl, rbase = nfl_rbase(t)
        clamp = nfl - 1
        gfin = (fvm[pl.ds(t * _NW * _CSTRIDE + wid * _CSTRIDE, 16)][0] + 15) >> 4

        def ng_of(b):
            return jnp.where(b == clamp, gfin, _K // 16)

        wait_idx(sb0, db0, semi0)
        start_gather(sb0, rw0, semg0)
        nit = nfl >> 1

        def it_body(k, carry2):
            wait_idx(sb1, db1, semi1)
            start_gather(sb1, rw1, semg1)
            wait_gather(rw0, semg0)
            accum(rw0, db0, ng_of(2 * k))
            start_idx(rbase + jnp.minimum(2 * k + 2, clamp) * _K, sb0, db0, semi0)
            wait_gather(rw1, semg1)
            accum(rw1, db1, ng_of(2 * k + 1))
            start_idx(rbase + jnp.minimum(2 * k + 3, clamp) * _K, sb1, db1, semi1)
            wait_idx(sb0, db0, semi0)
            start_gather(sb0, rw0, semg0)
            return carry2

        lax.fori_loop(0, nit, it_body, 0)
        wait_gather(rw0, semg0)
        tn = jnp.minimum(t + 1, _NW - 1)

        @pl.when((nfl & 1) == 1)
        def _():
            accum(rw0, db0, gfin)

        wait_idx(sb1, db1, semi1)
        start_idx(base_of(tn, 0), sb0, db0, semi0)
        start_idx(base_of(tn, 1), sb1, db1, semi1)
        return carry

    lax.fori_loop(0, _NW, src_body, 0)
    wait_idx(sb0, db0, semi0)
    wait_idx(sb1, db1, semi1)

    pltpu.sync_copy(acc.at[pl.ds(0, _NPW)], out_hbm.at[pl.ds(lo, _NPW)])


_sc_drain = functools.partial(
    pl.kernel,
    out_type=jax.ShapeDtypeStruct((_NPAD, _D), jnp.float32),
    mesh=plsc.VectorSubcoreMesh(core_axis_name="c", subcore_axis_name="s"),
    scratch_types=[
        pltpu.VMEM((_NPW + 1, _D), jnp.float32),
        pltpu.VMEM((_NW * _NW * _CSTRIDE,), jnp.int32),
        pltpu.VMEM((_NW * _NW * _CSTRIDE,), jnp.int32),
        pltpu.VMEM((_K,), jnp.int32),
        pltpu.VMEM((_K,), jnp.int32),
        pltpu.VMEM((_K + 16,), jnp.int32),
        pltpu.VMEM((_K + 16,), jnp.int32),
        pltpu.VMEM((_K, _D), jnp.float32),
        pltpu.VMEM((_K, _D), jnp.float32),
        pltpu.SemaphoreType.DMA,
        pltpu.SemaphoreType.DMA,
        pltpu.SemaphoreType.DMA,
        pltpu.SemaphoreType.DMA,
    ],
)(_sc_drain_body)


def _tc_body(h_ref, a_ref, wt_ref, b_ref, o_ref, *, relu):
    a = a_ref[...]
    x = h_ref[...] + jnp.where(a == -jnp.inf, 0.0, a)
    y = jnp.dot(x, wt_ref[...], preferred_element_type=jnp.float32) + b_ref[...]
    if relu:
        y = jnp.maximum(y, 0.0)
    o_ref[...] = y


def _tc_linear(h, agg, wt, b, relu):
    blk = 1000
    return pl.pallas_call(
        functools.partial(_tc_body, relu=relu),
        grid=(_N // blk,),
        in_specs=[
            pl.BlockSpec((blk, _D), lambda i: (i, 0)),
            pl.BlockSpec((blk, _D), lambda i: (i, 0)),
            pl.BlockSpec((_D, _D), lambda i: (0, 0)),
            pl.BlockSpec((1, _D), lambda i: (0, 0)),
        ],
        out_specs=pl.BlockSpec((blk, _D), lambda i: (i, 0)),
        out_shape=jax.ShapeDtypeStruct((_N, _D), jnp.float32),
    )(h, agg, wt, b.reshape(1, _D))


def kernel(h, edge_index, W1, b1, W2, b2):
    src = edge_index[0]
    dst = edge_index[1]
    sidx, didx, cnts, fills = _sc_bin(src, dst)
    agg1 = _sc_drain(h, sidx, didx, cnts, fills)
    h1 = _tc_linear(h, agg1[:_N], W1.T, b1, relu=True)
    agg2 = _sc_drain(h1, sidx, didx, cnts, fills)
    return _tc_linear(h1, agg2[:_N], W2.T, b2, relu=False)
